# Initial kernel scaffold; baseline (speedup 1.0000x reference)
import jax
import jax.numpy as jnp
from jax.experimental import pallas as pl


def kernel(dense_x, sparse_idx, emb_tables, W_d0, b_d0, W_d1, b_d1, W_d2, b_d2, W_c0, b_c0, W_c1, b_c1, W_c2, b_c2):
    raise NotImplementedError("write your pallas kernel here")


# speedup vs baseline: 2.0886x; 2.0886x over previous
import functools

import numpy as np
import jax
import jax.numpy as jnp
from jax import lax
from jax.experimental import pallas as pl
from jax.experimental.pallas import tpu as pltpu
from jax.experimental.pallas import tpu_sc as plsc

_B = 4096
_ND = 13
_NS = 26
_VOCAB = 100000
_D = 32
_NFM = _NS + 1
_NPAIR = (_NFM * (_NFM - 1)) // 2
_H0 = 512
_H1 = 256


_TOTAL_ROWS = _B * _NS
_IDX_MINOR = 128
_NW = 32
_ROWS_PER_W = _TOTAL_ROWS // _NW
_CHUNKS_PER_W = _ROWS_PER_W // _IDX_MINOR


def _sc_gather_body(table_hbm, idx_hbm, out_hbm, idx_v, rows_v, sem):
    nc = 2
    wid = lax.axis_index("s") * nc + lax.axis_index("c")
    pltpu.sync_copy(idx_hbm.at[pl.ds(wid * _ROWS_PER_W, _ROWS_PER_W)], idx_v)
    descs = []
    for c in range(_CHUNKS_PER_W):
        descs.append(
            pltpu.async_copy(
                table_hbm.at[idx_v.at[pl.ds(c * _IDX_MINOR, _IDX_MINOR)]],
                rows_v.at[pl.ds(c * _IDX_MINOR, _IDX_MINOR)],
                sem,
            )
        )
    for d in descs:
        d.wait()
    pltpu.sync_copy(rows_v, out_hbm.at[pl.ds(wid * _ROWS_PER_W, _ROWS_PER_W)])


def _sc_gather(emb_tables, idx1d):
    mesh = plsc.VectorSubcoreMesh(core_axis_name="c", subcore_axis_name="s")
    k = pl.kernel(
        _sc_gather_body,
        out_type=jax.ShapeDtypeStruct((_TOTAL_ROWS, _D), jnp.float32),
        mesh=mesh,
        scratch_types=[
            pltpu.VMEM((_ROWS_PER_W,), jnp.int32),
            pltpu.VMEM((_ROWS_PER_W, _D), jnp.float32),
            pltpu.SemaphoreType.DMA,
        ],
        compiler_params=pltpu.CompilerParams(use_tc_tiling_on_sc=False),
    )
    return k(emb_tables, idx1d)


_BLK = 512


def _tc_body(dense_ref, fm_ref,
             wd0_ref, bd0_ref, wd1_ref, bd1_ref, wd2_ref, bd2_ref,
             wff_ref, wfd_ref, wdd_ref, bc0_ref,
             wc1_ref, bc1_ref, wc2_ref, bc2_ref,
             out_ref):
    f32 = jnp.float32
    x = dense_ref[...]
    h = jnp.maximum(jnp.dot(x, wd0_ref[...], preferred_element_type=f32)
                    + bd0_ref[...], 0.0)
    h = jnp.maximum(jnp.dot(h, wd1_ref[...], preferred_element_type=f32)
                    + bd1_ref[...], 0.0)
    dense = jnp.maximum(jnp.dot(h, wd2_ref[...], preferred_element_type=f32)
                        + bd2_ref[...], 0.0)

    fm = fm_ref[...]
    z = lax.dot_general(fm, fm, (((2,), (2,)), ((0,), (0,))),
                        preferred_element_type=f32)
    zflat = z.reshape(_BLK, _NS * _NS)
    zfd = jnp.sum(fm * dense[:, None, :], axis=2)

    acc = jnp.dot(zflat, wff_ref[...], preferred_element_type=f32)
    acc = acc + jnp.dot(zfd, wfd_ref[...], preferred_element_type=f32)
    acc = acc + jnp.dot(dense, wdd_ref[...], preferred_element_type=f32)
    c = jnp.maximum(acc + bc0_ref[...], 0.0)
    c = jnp.maximum(jnp.dot(c, wc1_ref[...], preferred_element_type=f32)
                    + bc1_ref[...], 0.0)
    logit = jnp.dot(c, wc2_ref[...], preferred_element_type=f32) + bc2_ref[...]
    out_ref[...] = 1.0 / (1.0 + jnp.exp(-logit))


def _tc_forward(dense_x, fm3, wd0, bd0, wd1, bd1, wd2, bd2,
                wff, wfd, wdd, bc0, wc1, bc1, wc2, bc2):
    nblk = _B // _BLK
    full = lambda *shape: pl.BlockSpec(shape, lambda i: (0,) * len(shape))
    return pl.pallas_call(
        _tc_body,
        grid=(nblk,),
        in_specs=[
            pl.BlockSpec((_BLK, _ND), lambda i: (i, 0)),
            pl.BlockSpec((_BLK, _NS, _D), lambda i: (i, 0, 0)),
            full(_ND, _H0), full(1, _H0),
            full(_H0, _H1), full(1, _H1),
            full(_H1, _D), full(1, _D),
            full(_NS * _NS, _H0), full(_NS, _H0), full(_D, _H0), full(1, _H0),
            full(_H0, _H1), full(1, _H1),
            full(_H1, 1), full(1, 1),
        ],
        out_specs=pl.BlockSpec((_BLK, 1), lambda i: (i, 0)),
        out_shape=jax.ShapeDtypeStruct((_B, 1), jnp.float32),
        compiler_params=pltpu.CompilerParams(
            dimension_semantics=("arbitrary",),
        ),
    )(dense_x, fm3, wd0, bd0, wd1, bd1, wd2, bd2,
      wff, wfd, wdd, bc0, wc1, bc1, wc2, bc2)


_IU0, _IU1 = np.triu_indices(_NFM, 1)
_FF_MASK = (_IU0 < _NS) & (_IU1 < _NS)
_FF_DEST = (_IU0[_FF_MASK] * _NS + _IU1[_FF_MASK]).astype(np.int32)
_FF_SRC = np.nonzero(_FF_MASK)[0].astype(np.int32)
_FD_SRC = np.nonzero(~_FF_MASK)[0].astype(np.int32)


def kernel(dense_x, sparse_idx, emb_tables,
           W_d0, b_d0, W_d1, b_d1, W_d2, b_d2,
           W_c0, b_c0, W_c1, b_c1, W_c2, b_c2):
    offsets = jnp.arange(_NS, dtype=sparse_idx.dtype) * _VOCAB
    flat_idx = (sparse_idx + offsets[None, :]).reshape(-1)
    fm_flat = _sc_gather(emb_tables, flat_idx)
    fm3 = fm_flat.reshape(_B, _NS, _D)

    wff = jnp.zeros((_NS * _NS, _H0), jnp.float32).at[_FF_DEST].set(
        W_c0[_FF_SRC])
    wfd = W_c0[_FD_SRC]
    wdd = W_c0[_NPAIR:]

    return _tc_forward(
        dense_x, fm3,
        W_d0, b_d0.reshape(1, -1), W_d1, b_d1.reshape(1, -1),
        W_d2, b_d2.reshape(1, -1),
        wff, wfd, wdd, b_c0.reshape(1, -1),
        W_c1, b_c1.reshape(1, -1), W_c2, b_c2.reshape(1, -1))
